# baseline (speedup 1.0000x reference)
import functools

import jax
import jax.numpy as jnp
from jax import lax
from jax.experimental import pallas as pl
from jax.experimental.pallas import tpu as pltpu
from jax.experimental.pallas import tpu_sc as plsc

_N = 10000
_D = 128
_E = 320000

_CHUNK = 80
_NCHUNKS = _E // _CHUNK
_NWORKERS = 32
_NSLOT = _NCHUNKS // _NWORKERS
_NIDX = 8
_NROW = 4
_PIECES = tuple((k * _CHUNK, _CHUNK) for k in range(7))


def _sc_scatter_body(h_hbm, sd_hbm, out_hbm,
                     ib0, ib1, ib2, ib3, ib4, ib5, ib6, ib7,
                     rows0, rows1, rows2, rows3, acc,
                     is0, is1, is2, is3, is4, is5, is6, is7,
                     gs0, gs1, gs2, gs3, ts0, ts1, ts2, ts3):
    ib = (ib0, ib1, ib2, ib3, ib4, ib5, ib6, ib7)
    rows = (rows0, rows1, rows2, rows3)
    isem = (is0, is1, is2, is3, is4, is5, is6, is7)
    gsem = (gs0, gs1, gs2, gs3)
    ssem = (ts0, ts1, ts2, ts3)
    cid = lax.axis_index("c")
    sid = lax.axis_index("s")
    wid = sid * 2 + cid

    def _zero_row(i, carry):
        for j in range(_D // 16):
            rows0[i, pl.ds(j * 16, 16)] = jnp.zeros((16,), jnp.float32)
        return carry
    lax.fori_loop(0, _CHUNK, _zero_row, 0)
    row0 = pl.multiple_of(sid * 624 + lax.select(sid == 15, 8, 0), 8)
    for (off, sz) in _PIECES:
        pltpu.async_copy(rows0.at[pl.ds(0, sz)],
                         acc.at[pl.ds(row0 + off, sz)], gs0)

    @pl.when(sid < 14)
    def _():
        pltpu.async_copy(rows0.at[pl.ds(0, 64)],
                        acc.at[pl.ds(row0 + 560, 64)], gs1)

    @pl.when(sid >= 14)
    def _():
        pltpu.async_copy(rows0.at[pl.ds(0, 72)],
                        acc.at[pl.ds(row0 + 560, 72)], gs1)

    def _ph_i(s, m8):
        pltpu.async_copy(sd_hbm.at[wid + s * _NWORKERS], ib[m8], isem[m8])

    def _ph_a(m8, m4):
        pltpu.make_async_copy(sd_hbm.at[0], ib[m8], isem[m8]).wait()
        pltpu.async_copy(h_hbm.at[ib[m8].at[0]], rows[m4], gsem[m4])

    def _ph_b(m8, m4):
        pltpu.make_async_copy(h_hbm.at[ib[m8].at[0]], rows[m4],
                              gsem[m4]).wait()
        pltpu.async_copy(rows[m4], acc.at[ib[m8].at[1]], ssem[m4], add=True)

    def _drain(m8, m4):
        pltpu.make_async_copy(rows[m4], acc.at[ib[m8].at[1]], ssem[m4]).wait()

    _ph_i(0, 0)
    _ph_i(1, 1)
    for (off, sz) in _PIECES:
        pltpu.make_async_copy(rows0.at[pl.ds(0, sz)],
                              acc.at[pl.ds(row0 + off, sz)], gs0).wait()

    @pl.when(sid < 14)
    def _():
        pltpu.make_async_copy(rows0.at[pl.ds(0, 64)],
                              acc.at[pl.ds(row0 + 560, 64)], gs1).wait()

    @pl.when(sid >= 14)
    def _():
        pltpu.make_async_copy(rows0.at[pl.ds(0, 72)],
                              acc.at[pl.ds(row0 + 560, 72)], gs1).wait()
    plsc.subcore_barrier()

    def _outer(k, carry):
        j0 = k * _NIDX
        for m in range(_NIDX):
            j = j0 + m

            @pl.when(j <= _NSLOT - 3)
            def _(m=m, j=j):
                _ph_i(j + 2, (m + 2) % _NIDX)

            @pl.when((j >= 2) & (j <= _NSLOT + 1))
            def _(m=m):
                _ph_b((m - 2) % _NIDX, (m - 2) % _NROW)

            @pl.when(j >= _NROW)
            def _(m=m):
                _drain((m - _NROW) % _NIDX, m % _NROW)

            @pl.when(j <= _NSLOT - 1)
            def _(m=m):
                _ph_a(m, m % _NROW)
        return carry
    lax.fori_loop(0, -(-(_NSLOT + 3) // _NIDX), _outer, 0)
    _drain((_NSLOT - 1) % _NIDX, (_NSLOT - 1) % _NROW)
    plsc.subcore_barrier()

    def _wb_rd(k, b):
        return pltpu.make_async_copy(
            acc.at[pl.ds(pl.multiple_of(row0 + k * _CHUNK, 8), _CHUNK)],
            rows[b], gsem[b])

    def _wb_wr(k, b):
        return pltpu.make_async_copy(
            rows[b],
            out_hbm.at[cid, pl.ds(pl.multiple_of(row0 + k * _CHUNK, 8),
                                  _CHUNK)], ssem[b])

    for k in range(4):
        _wb_rd(k, k).start()
    for k in range(7):
        b = k % 4
        _wb_rd(k, b).wait()
        _wb_wr(k, b).start()
        if k + 4 <= 6:
            _wb_wr(k, b).wait()
            _wb_rd(k + 4, b).start()
    _wb_wr(3, 3).wait()

    @pl.when(sid < 14)
    def _():
        r = pl.multiple_of(row0 + 560, 8)
        bb = rows3.at[pl.ds(0, 64)]
        pltpu.sync_copy(acc.at[pl.ds(r, 64)], bb)
        pltpu.sync_copy(bb, out_hbm.at[cid, pl.ds(r, 64)])

    @pl.when(sid >= 14)
    def _():
        r = pl.multiple_of(row0 + 560, 8)
        bb = rows3.at[pl.ds(0, 72)]
        pltpu.sync_copy(acc.at[pl.ds(r, 72)], bb)
        pltpu.sync_copy(bb, out_hbm.at[cid, pl.ds(r, 72)])

    for b in range(3):
        _wb_wr(b + 4, b).wait()


@functools.cache
def _make_sc_scatter():
    return pl.kernel(
        _sc_scatter_body,
        out_type=jax.ShapeDtypeStruct((2, _N, _D), jnp.float32),
        mesh=plsc.VectorSubcoreMesh(core_axis_name="c", subcore_axis_name="s"),
        scratch_types=(
            [pltpu.VMEM((2, _CHUNK), jnp.int32)] * _NIDX
            + [pltpu.VMEM((_CHUNK, _D), jnp.float32)] * _NROW
            + [pltpu.VMEM_SHARED((_N, _D), jnp.float32)]
            + [pltpu.SemaphoreType.DMA] * (_NIDX + 2 * _NROW)
        ),
    )


def _sc_scatter(h, sd):
    return _make_sc_scatter()(h, sd)


_R = 5000


def _pre_body(x_ref, w1, b1, w2, b2, o_ref):
    h = jnp.maximum(
        jnp.dot(x_ref[...], w1[...], preferred_element_type=jnp.float32)
        + b1[...], 0.0)
    o_ref[...] = (jnp.dot(h, w2[...], preferred_element_type=jnp.float32)
                  + b2[...])


def _conv_body(h_ref, agg_ref, w1, b1, w2, b2, o_ref):
    u = h_ref[...] + agg_ref[0] + agg_ref[1]
    h = jnp.maximum(
        jnp.dot(u, w1[...], preferred_element_type=jnp.float32) + b1[...], 0.0)
    o_ref[...] = (jnp.dot(h, w2[...], preferred_element_type=jnp.float32)
                  + b2[...])


def _post_body(x_ref, p0, p1, p2, p3, w1, b1, ln_g, ln_b, w2, b2, o_ref):
    z = jnp.dot(x_ref[...], w1[0], preferred_element_type=jnp.float32)
    z += jnp.dot(p0[...], w1[1], preferred_element_type=jnp.float32)
    z += jnp.dot(p1[...], w1[2], preferred_element_type=jnp.float32)
    z += jnp.dot(p2[...], w1[3], preferred_element_type=jnp.float32)
    z += jnp.dot(p3[...], w1[4], preferred_element_type=jnp.float32)
    z += b1[...]
    m = jnp.mean(z, axis=-1, keepdims=True)
    var = jnp.mean((z - m) * (z - m), axis=-1, keepdims=True)
    zn = (z - m) / jnp.sqrt(var + 1e-5) * ln_g[...] + ln_b[...]
    h = jnp.maximum(zn, 0.0)
    o_ref[...] = (jnp.dot(h, w2[...], preferred_element_type=jnp.float32)
                  + b2[...])


def _row_spec(r=_R):
    return pl.BlockSpec((r, _D), lambda i: (i, 0))


def _full_spec(shape):
    nd = len(shape)
    return pl.BlockSpec(shape, lambda i: (0,) * nd)


def _pre_mlp(x, w1, b1, w2, b2):
    return pl.pallas_call(
        _pre_body,
        grid=(_N // _R,),
        in_specs=[_row_spec(), _full_spec((_D, _D)), _full_spec((1, _D)),
                  _full_spec((_D, _D)), _full_spec((1, _D))],
        out_specs=_row_spec(),
        out_shape=jax.ShapeDtypeStruct((_N, _D), jnp.float32),
    )(x, w1, b1, w2, b2)


def _conv_mlp(h, agg2, w1, b1, w2, b2):
    return pl.pallas_call(
        _conv_body,
        grid=(_N // _R,),
        in_specs=[_row_spec(),
                  pl.BlockSpec((2, _R, _D), lambda i: (0, i, 0)),
                  _full_spec((_D, _D)), _full_spec((1, _D)),
                  _full_spec((_D, _D)), _full_spec((1, _D))],
        out_specs=_row_spec(),
        out_shape=jax.ShapeDtypeStruct((_N, _D), jnp.float32),
    )(h, agg2, w1, b1, w2, b2)


def _post_mlp(x, h0, h1, h2, h3, w1s, b1, ln_g, ln_b, w2, b2):
    return pl.pallas_call(
        _post_body,
        grid=(_N // _R,),
        in_specs=[_row_spec(), _row_spec(), _row_spec(), _row_spec(),
                  _row_spec(),
                  _full_spec((5, _D, _D)), _full_spec((1, _D)),
                  _full_spec((1, _D)), _full_spec((1, _D)),
                  _full_spec((_D, _D)), _full_spec((1, _D))],
        out_specs=_row_spec(),
        out_shape=jax.ShapeDtypeStruct((_N, _D), jnp.float32),
    )(x, h0, h1, h2, h3, w1s, b1, ln_g, ln_b, w2, b2)


def kernel(x, edge_index, pre_W1, pre_b1, pre_W2, pre_b2,
           c0_W1, c0_b1, c0_W2, c0_b2,
           c1_W1, c1_b1, c1_W2, c1_b2,
           c2_W1, c2_b1, c2_W2, c2_b2,
           post_W1, post_b1, post_ln_g, post_ln_b, post_W2, post_b2):
    sd = jnp.stack([edge_index[0].reshape(_NCHUNKS, _CHUNK),
                    edge_index[1].reshape(_NCHUNKS, _CHUNK)], axis=1)
    row = lambda b: b.reshape(1, _D)

    h = _pre_mlp(x, pre_W1, row(pre_b1), pre_W2, row(pre_b2))
    jump = [x, h]
    for (w1, b1, w2, b2) in ((c0_W1, c0_b1, c0_W2, c0_b2),
                             (c1_W1, c1_b1, c1_W2, c1_b2),
                             (c2_W1, c2_b1, c2_W2, c2_b2)):
        agg2 = _sc_scatter(h, sd)
        h = _conv_mlp(h, agg2, w1, row(b1), w2, row(b2))
        jump.append(h)

    w1s = post_W1.reshape(5, _D, _D)
    return _post_mlp(jump[0], jump[1], jump[2], jump[3], jump[4],
                     w1s, row(post_b1), row(post_ln_g), row(post_ln_b),
                     post_W2, row(post_b2))


# speedup vs baseline: 1.0169x; 1.0169x over previous
import functools

import jax
import jax.numpy as jnp
from jax import lax
from jax.experimental import pallas as pl
from jax.experimental.pallas import tpu as pltpu
from jax.experimental.pallas import tpu_sc as plsc

_N = 10000
_D = 128
_E = 320000

_CHUNK = 80
_NCHUNKS = _E // _CHUNK
_NWORKERS = 32
_NSLOT = _NCHUNKS // _NWORKERS
_NIDX = 8
_NROW = 4
_PIECES = tuple((k * _CHUNK, _CHUNK) for k in range(7))


def _sc_scatter_body(h_hbm, src_hbm, dst_hbm, out_hbm,
                     si0, si1, si2, si3, si4, si5, si6, si7,
                     di0, di1, di2, di3, di4, di5, di6, di7,
                     rows0, rows1, rows2, rows3, acc,
                     is0, is1, is2, is3, is4, is5, is6, is7,
                     gs0, gs1, gs2, gs3, ts0, ts1, ts2, ts3):
    sidx = (si0, si1, si2, si3, si4, si5, si6, si7)
    didx = (di0, di1, di2, di3, di4, di5, di6, di7)
    rows = (rows0, rows1, rows2, rows3)
    isem = (is0, is1, is2, is3, is4, is5, is6, is7)
    gsem = (gs0, gs1, gs2, gs3)
    ssem = (ts0, ts1, ts2, ts3)
    cid = lax.axis_index("c")
    sid = lax.axis_index("s")
    wid = sid * 2 + cid

    def _zero_row(i, carry):
        for j in range(_D // 16):
            rows0[i, pl.ds(j * 16, 16)] = jnp.zeros((16,), jnp.float32)
        return carry
    lax.fori_loop(0, _CHUNK, _zero_row, 0)
    row0 = pl.multiple_of(sid * 624 + lax.select(sid == 15, 8, 0), 8)
    for (off, sz) in _PIECES:
        pltpu.async_copy(rows0.at[pl.ds(0, sz)],
                         acc.at[pl.ds(row0 + off, sz)], gs0)

    @pl.when(sid < 14)
    def _():
        pltpu.async_copy(rows0.at[pl.ds(0, 64)],
                        acc.at[pl.ds(row0 + 560, 64)], gs1)

    @pl.when(sid >= 14)
    def _():
        pltpu.async_copy(rows0.at[pl.ds(0, 72)],
                        acc.at[pl.ds(row0 + 560, 72)], gs1)

    def _ph_i(s, m8):
        base = pl.multiple_of((wid + s * _NWORKERS) * _CHUNK, 16)
        pltpu.async_copy(src_hbm.at[pl.ds(base, _CHUNK)], sidx[m8], isem[m8])
        pltpu.async_copy(dst_hbm.at[pl.ds(base, _CHUNK)], didx[m8], isem[m8])

    def _ph_a(m8, m4):
        pltpu.make_async_copy(src_hbm.at[pl.ds(0, _CHUNK)], sidx[m8],
                              isem[m8]).wait()
        pltpu.make_async_copy(dst_hbm.at[pl.ds(0, _CHUNK)], didx[m8],
                              isem[m8]).wait()
        pltpu.async_copy(h_hbm.at[sidx[m8]], rows[m4], gsem[m4])

    def _ph_b(m8, m4):
        pltpu.make_async_copy(h_hbm.at[sidx[m8]], rows[m4], gsem[m4]).wait()
        pltpu.async_copy(rows[m4], acc.at[didx[m8]], ssem[m4], add=True)

    def _drain(m8, m4):
        pltpu.make_async_copy(rows[m4], acc.at[didx[m8]], ssem[m4]).wait()

    _ph_i(0, 0)
    _ph_i(1, 1)
    for (off, sz) in _PIECES:
        pltpu.make_async_copy(rows0.at[pl.ds(0, sz)],
                              acc.at[pl.ds(row0 + off, sz)], gs0).wait()

    @pl.when(sid < 14)
    def _():
        pltpu.make_async_copy(rows0.at[pl.ds(0, 64)],
                              acc.at[pl.ds(row0 + 560, 64)], gs1).wait()

    @pl.when(sid >= 14)
    def _():
        pltpu.make_async_copy(rows0.at[pl.ds(0, 72)],
                              acc.at[pl.ds(row0 + 560, 72)], gs1).wait()
    plsc.subcore_barrier()

    def _outer(k, carry):
        j0 = k * _NIDX
        for m in range(_NIDX):
            j = j0 + m

            @pl.when(j <= _NSLOT - 3)
            def _(m=m, j=j):
                _ph_i(j + 2, (m + 2) % _NIDX)

            @pl.when((j >= 2) & (j <= _NSLOT + 1))
            def _(m=m):
                _ph_b((m - 2) % _NIDX, (m - 2) % _NROW)

            @pl.when(j >= _NROW)
            def _(m=m):
                _drain((m - _NROW) % _NIDX, m % _NROW)

            @pl.when(j <= _NSLOT - 1)
            def _(m=m):
                _ph_a(m, m % _NROW)
        return carry
    lax.fori_loop(0, -(-(_NSLOT + 3) // _NIDX), _outer, 0)
    _drain((_NSLOT - 1) % _NIDX, (_NSLOT - 1) % _NROW)
    plsc.subcore_barrier()

    def _wb_rd(k, b):
        return pltpu.make_async_copy(
            acc.at[pl.ds(pl.multiple_of(row0 + k * _CHUNK, 8), _CHUNK)],
            rows[b], gsem[b])

    def _wb_wr(k, b):
        return pltpu.make_async_copy(
            rows[b],
            out_hbm.at[cid, pl.ds(pl.multiple_of(row0 + k * _CHUNK, 8),
                                  _CHUNK)], ssem[b])

    for k in range(4):
        _wb_rd(k, k).start()
    for k in range(7):
        b = k % 4
        _wb_rd(k, b).wait()
        _wb_wr(k, b).start()
        if k + 4 <= 6:
            _wb_wr(k, b).wait()
            _wb_rd(k + 4, b).start()
    _wb_wr(3, 3).wait()

    @pl.when(sid < 14)
    def _():
        r = pl.multiple_of(row0 + 560, 8)
        bb = rows3.at[pl.ds(0, 64)]
        pltpu.sync_copy(acc.at[pl.ds(r, 64)], bb)
        pltpu.sync_copy(bb, out_hbm.at[cid, pl.ds(r, 64)])

    @pl.when(sid >= 14)
    def _():
        r = pl.multiple_of(row0 + 560, 8)
        bb = rows3.at[pl.ds(0, 72)]
        pltpu.sync_copy(acc.at[pl.ds(r, 72)], bb)
        pltpu.sync_copy(bb, out_hbm.at[cid, pl.ds(r, 72)])

    for b in range(3):
        _wb_wr(b + 4, b).wait()


@functools.cache
def _make_sc_scatter():
    return pl.kernel(
        _sc_scatter_body,
        out_type=jax.ShapeDtypeStruct((2, _N, _D), jnp.float32),
        mesh=plsc.VectorSubcoreMesh(core_axis_name="c", subcore_axis_name="s"),
        scratch_types=(
            [pltpu.VMEM((_CHUNK,), jnp.int32)] * _NIDX
            + [pltpu.VMEM((_CHUNK,), jnp.int32)] * _NIDX
            + [pltpu.VMEM((_CHUNK, _D), jnp.float32)] * _NROW
            + [pltpu.VMEM_SHARED((_N, _D), jnp.float32)]
            + [pltpu.SemaphoreType.DMA] * (_NIDX + 2 * _NROW)
        ),
    )


def _sc_scatter(h, src, dst):
    return _make_sc_scatter()(h, src, dst)


_R = 5000


def _pre_body(x_ref, w1, b1, w2, b2, o_ref):
    h = jnp.maximum(
        jnp.dot(x_ref[...], w1[...], preferred_element_type=jnp.float32)
        + b1[...], 0.0)
    o_ref[...] = (jnp.dot(h, w2[...], preferred_element_type=jnp.float32)
                  + b2[...])


def _conv_body(h_ref, agg_ref, w1, b1, w2, b2, o_ref):
    u = h_ref[...] + agg_ref[0] + agg_ref[1]
    h = jnp.maximum(
        jnp.dot(u, w1[...], preferred_element_type=jnp.float32) + b1[...], 0.0)
    o_ref[...] = (jnp.dot(h, w2[...], preferred_element_type=jnp.float32)
                  + b2[...])


def _post_body(x_ref, p0, p1, p2, p3, w1, b1, ln_g, ln_b, w2, b2, o_ref):
    z = jnp.dot(x_ref[...], w1[0], preferred_element_type=jnp.float32)
    z += jnp.dot(p0[...], w1[1], preferred_element_type=jnp.float32)
    z += jnp.dot(p1[...], w1[2], preferred_element_type=jnp.float32)
    z += jnp.dot(p2[...], w1[3], preferred_element_type=jnp.float32)
    z += jnp.dot(p3[...], w1[4], preferred_element_type=jnp.float32)
    z += b1[...]
    m = jnp.mean(z, axis=-1, keepdims=True)
    var = jnp.mean((z - m) * (z - m), axis=-1, keepdims=True)
    zn = (z - m) / jnp.sqrt(var + 1e-5) * ln_g[...] + ln_b[...]
    h = jnp.maximum(zn, 0.0)
    o_ref[...] = (jnp.dot(h, w2[...], preferred_element_type=jnp.float32)
                  + b2[...])


def _row_spec(r=_R):
    return pl.BlockSpec((r, _D), lambda i: (i, 0))


def _full_spec(shape):
    nd = len(shape)
    return pl.BlockSpec(shape, lambda i: (0,) * nd)


def _pre_mlp(x, w1, b1, w2, b2):
    return pl.pallas_call(
        _pre_body,
        grid=(_N // _R,),
        in_specs=[_row_spec(), _full_spec((_D, _D)), _full_spec((1, _D)),
                  _full_spec((_D, _D)), _full_spec((1, _D))],
        out_specs=_row_spec(),
        out_shape=jax.ShapeDtypeStruct((_N, _D), jnp.float32),
    )(x, w1, b1, w2, b2)


def _conv_mlp(h, agg2, w1, b1, w2, b2):
    return pl.pallas_call(
        _conv_body,
        grid=(_N // _R,),
        in_specs=[_row_spec(),
                  pl.BlockSpec((2, _R, _D), lambda i: (0, i, 0)),
                  _full_spec((_D, _D)), _full_spec((1, _D)),
                  _full_spec((_D, _D)), _full_spec((1, _D))],
        out_specs=_row_spec(),
        out_shape=jax.ShapeDtypeStruct((_N, _D), jnp.float32),
    )(h, agg2, w1, b1, w2, b2)


def _post_mlp(x, h0, h1, h2, h3, w1s, b1, ln_g, ln_b, w2, b2):
    return pl.pallas_call(
        _post_body,
        grid=(_N // _R,),
        in_specs=[_row_spec(), _row_spec(), _row_spec(), _row_spec(),
                  _row_spec(),
                  _full_spec((5, _D, _D)), _full_spec((1, _D)),
                  _full_spec((1, _D)), _full_spec((1, _D)),
                  _full_spec((_D, _D)), _full_spec((1, _D))],
        out_specs=_row_spec(),
        out_shape=jax.ShapeDtypeStruct((_N, _D), jnp.float32),
    )(x, h0, h1, h2, h3, w1s, b1, ln_g, ln_b, w2, b2)


def kernel(x, edge_index, pre_W1, pre_b1, pre_W2, pre_b2,
           c0_W1, c0_b1, c0_W2, c0_b2,
           c1_W1, c1_b1, c1_W2, c1_b2,
           c2_W1, c2_b1, c2_W2, c2_b2,
           post_W1, post_b1, post_ln_g, post_ln_b, post_W2, post_b2):
    src = edge_index[0]
    dst = edge_index[1]
    row = lambda b: b.reshape(1, _D)

    h = _pre_mlp(x, pre_W1, row(pre_b1), pre_W2, row(pre_b2))
    jump = [x, h]
    for (w1, b1, w2, b2) in ((c0_W1, c0_b1, c0_W2, c0_b2),
                             (c1_W1, c1_b1, c1_W2, c1_b2),
                             (c2_W1, c2_b1, c2_W2, c2_b2)):
        agg2 = _sc_scatter(h, src, dst)
        h = _conv_mlp(h, agg2, w1, row(b1), w2, row(b2))
        jump.append(h)

    w1s = post_W1.reshape(5, _D, _D)
    return _post_mlp(jump[0], jump[1], jump[2], jump[3], jump[4],
                     w1s, row(post_b1), row(post_ln_g), row(post_ln_b),
                     post_W2, row(post_b2))
